# baseline (speedup 1.0000x reference)
import functools

import jax
import jax.numpy as jnp
from jax import lax
from jax.experimental import pallas as pl
from jax.experimental.pallas import tpu as pltpu
from jax.experimental.pallas import tpu_sc as plsc

N_SPEC = 1024
B = 1024
R1_N = 4096
R2_N = 16384
RBLK = 4096
NBLK2 = R2_N // RBLK
NC = 2
NS = 16
NW = NC * NS
BPW = B // NW
L = 16
YW = N_SPEC * BPW
AW = N_SPEC * BPW


def _build_sc_kernel():
    mesh = plsc.VectorSubcoreMesh(core_axis_name="c", subcore_axis_name="s")

    @functools.partial(
        pl.kernel,
        mesh=mesh,
        out_type=jax.ShapeDtypeStruct((NW, AW), jnp.float32),
        scratch_types=[
            pltpu.VMEM((YW,), jnp.float32),
            pltpu.VMEM((AW,), jnp.float32),
            pltpu.VMEM((RBLK,), jnp.int32),
            pltpu.VMEM((RBLK,), jnp.float32),
            pltpu.VMEM((RBLK,), jnp.float32),
            pltpu.VMEM((BPW,), jnp.float32),
        ],
    )
    def reaction_kernel(yr_hbm, t_hbm, pk1_hbm, a1_hbm, e1_hbm,
                        pk2_hbm, a2_hbm, e2_hbm,
                        out_hbm, y_v, acc_v, pk_v, a_v, e_v, t_v):
        wid = lax.axis_index("s") * NC + lax.axis_index("c")

        pltpu.sync_copy(yr_hbm.at[wid], y_v)
        pltpu.sync_copy(t_hbm.at[pl.ds(wid * BPW, BPW)], t_v)

        def zero_body(s, carry):
            acc_v[pl.ds(s * L, L)] = jnp.zeros((L,), jnp.float32)
            return carry
        lax.fori_loop(0, AW // L, zero_body, 0)

        invt0 = 1.0 / t_v[pl.ds(0, L)]
        invt1 = 1.0 / t_v[pl.ds(L, L)]

        pltpu.sync_copy(pk1_hbm, pk_v)
        pltpu.sync_copy(a1_hbm, a_v)
        pltpu.sync_copy(e1_hbm, e_v)

        def body1(g, carry):
            it0, it1 = carry
            gb = g * L
            pkv16 = pk_v[pl.ds(gb, L)]
            av16 = a_v[pl.ds(gb, L)]
            ev16 = e_v[pl.ds(gb, L)]
            for k in range(L):
                pk = pkv16[k]
                i = (pk & 1023) << 5
                p = (pk >> 10) << 5
                a = av16[k]
                ep = ev16[k]
                yi0 = y_v[pl.ds(i, L)]
                term0 = yi0 * (a * jnp.exp(ep * it0))
                plsc.addupdate(acc_v.at[pl.ds(p, L)], term0)
                yi1 = y_v[pl.ds(i + L, L)]
                term1 = yi1 * (a * jnp.exp(ep * it1))
                plsc.addupdate(acc_v.at[pl.ds(p + L, L)], term1)
            return carry
        lax.fori_loop(0, R1_N // L, body1, (invt0, invt1))

        for blk in range(NBLK2):
            base = blk * RBLK
            pltpu.sync_copy(pk2_hbm.at[pl.ds(base, RBLK)], pk_v)
            pltpu.sync_copy(a2_hbm.at[pl.ds(base, RBLK)], a_v)
            pltpu.sync_copy(e2_hbm.at[pl.ds(base, RBLK)], e_v)

            def body2(g, carry):
                it0, it1 = carry
                gb = g * (2 * L)
                for h in range(2):
                    hb = gb + h * L
                    pkv16 = pk_v[pl.ds(hb, L)]
                    av16 = a_v[pl.ds(hb, L)]
                    ev16 = e_v[pl.ds(hb, L)]
                    for k in range(L):
                        pk = pkv16[k]
                        i = (pk & 1023) << 5
                        j = ((pk >> 10) & 1023) << 5
                        p = (pk >> 20) << 5
                        a = av16[k]
                        ep = ev16[k]
                        yi0 = y_v[pl.ds(i, L)]
                        yj0 = y_v[pl.ds(j, L)]
                        term0 = (yi0 * yj0) * (a * jnp.exp(ep * it0))
                        plsc.addupdate(acc_v.at[pl.ds(p, L)], term0)
                        yi1 = y_v[pl.ds(i + L, L)]
                        yj1 = y_v[pl.ds(j + L, L)]
                        term1 = (yi1 * yj1) * (a * jnp.exp(ep * it1))
                        plsc.addupdate(acc_v.at[pl.ds(p + L, L)], term1)
                return carry
            lax.fori_loop(0, RBLK // (2 * L), body2, (invt0, invt1))

        pltpu.sync_copy(acc_v, out_hbm.at[wid])

    return reaction_kernel


_SC_KERNEL = _build_sc_kernel()


def kernel(t_in, y_in, inds_1r, inds_1p, inds_2r, inds_2p, A1, E1, A2, E2):
    pk1 = inds_1r.astype(jnp.int32) | (inds_1p.astype(jnp.int32) << 10)
    ep1 = -E1
    pk2 = (inds_2r[:, 0].astype(jnp.int32)
           | (inds_2r[:, 1].astype(jnp.int32) << 10)
           | (inds_2p.astype(jnp.int32) << 20))
    ep2 = -E2
    yr = y_in.reshape(NW, BPW, N_SPEC).transpose(0, 2, 1).reshape(NW, YW)
    tflat = t_in.reshape(B)

    out = _SC_KERNEL(yr, tflat, pk1, A1, ep1, pk2, A2, ep2)
    return out.reshape(NW, N_SPEC, BPW).transpose(0, 2, 1).reshape(B, N_SPEC)
